# baseline (speedup 1.0000x reference)
import jax
import jax.numpy as jnp
from jax.experimental import pallas as pl
from jax.experimental.pallas import tpu as pltpu


def kernel(x, embedding):
    seq_len = x.shape[1]
    d_model = embedding.shape[-1]

    def body(emb_ref, out_ref, sem):
        copy = pltpu.make_async_copy(
            emb_ref.at[:, pl.ds(0, seq_len), :], out_ref, sem
        )
        copy.start()
        copy.wait()

    return pl.pallas_call(
        body,
        in_specs=[pl.BlockSpec(memory_space=pl.ANY)],
        out_specs=pl.BlockSpec(memory_space=pl.ANY),
        out_shape=jax.ShapeDtypeStruct((1, seq_len, d_model), embedding.dtype),
        scratch_shapes=[pltpu.SemaphoreType.DMA],
    )(embedding)


# speedup vs baseline: 29.2124x; 29.2124x over previous
import jax
import jax.numpy as jnp
from jax.experimental import pallas as pl
from jax.experimental.pallas import tpu as pltpu

_BLOCK = 256


def _copy_body(emb_ref, out_ref):
    out_ref[...] = emb_ref[...]


def kernel(x, embedding):
    seq_len = x.shape[1]
    d_model = embedding.shape[-1]
    block = min(_BLOCK, seq_len)
    grid = (seq_len + block - 1) // block

    return pl.pallas_call(
        _copy_body,
        grid=(grid,),
        in_specs=[
            pl.BlockSpec((1, block, d_model), lambda i: (0, i, 0)),
        ],
        out_specs=pl.BlockSpec((1, block, d_model), lambda i: (0, i, 0)),
        out_shape=jax.ShapeDtypeStruct((1, seq_len, d_model), embedding.dtype),
        compiler_params=pltpu.CompilerParams(
            dimension_semantics=("parallel",),
        ),
    )(embedding)


# speedup vs baseline: 38.7114x; 1.3252x over previous
import jax
import jax.numpy as jnp
from jax.experimental import pallas as pl
from jax.experimental.pallas import tpu as pltpu

_BLOCK = 512


def _copy_body(emb_ref, out_ref):
    out_ref[...] = emb_ref[...]


def kernel(x, embedding):
    seq_len = x.shape[1]
    d_model = embedding.shape[-1]
    block = min(_BLOCK, seq_len)
    grid = (seq_len + block - 1) // block

    return pl.pallas_call(
        _copy_body,
        grid=(grid,),
        in_specs=[
            pl.BlockSpec((1, block, d_model), lambda i: (0, i, 0)),
        ],
        out_specs=pl.BlockSpec((1, block, d_model), lambda i: (0, i, 0)),
        out_shape=jax.ShapeDtypeStruct((1, seq_len, d_model), embedding.dtype),
        compiler_params=pltpu.CompilerParams(
            dimension_semantics=("parallel",),
        ),
    )(embedding)


# speedup vs baseline: 41.4446x; 1.0706x over previous
import jax
import jax.numpy as jnp
from jax.experimental import pallas as pl
from jax.experimental.pallas import tpu as pltpu

_BLOCK = 1024


def _copy_body(emb_ref, out_ref):
    out_ref[...] = emb_ref[...]


def kernel(x, embedding):
    seq_len = x.shape[1]
    d_model = embedding.shape[-1]
    block = min(_BLOCK, seq_len)
    grid = (seq_len + block - 1) // block

    return pl.pallas_call(
        _copy_body,
        grid=(grid,),
        in_specs=[
            pl.BlockSpec((1, block, d_model), lambda i: (0, i, 0)),
        ],
        out_specs=pl.BlockSpec((1, block, d_model), lambda i: (0, i, 0)),
        out_shape=jax.ShapeDtypeStruct((1, seq_len, d_model), embedding.dtype),
        compiler_params=pltpu.CompilerParams(
            dimension_semantics=("parallel",),
        ),
    )(embedding)


# speedup vs baseline: 47.4275x; 1.1444x over previous
import jax
import jax.numpy as jnp
from jax.experimental import pallas as pl
from jax.experimental.pallas import tpu as pltpu

_BLOCK = 2048


def _copy_body(emb_ref, out_ref):
    out_ref[...] = emb_ref[...]


def kernel(x, embedding):
    seq_len = x.shape[1]
    d_model = embedding.shape[-1]
    block = min(_BLOCK, seq_len)
    grid = (seq_len + block - 1) // block

    return pl.pallas_call(
        _copy_body,
        grid=(grid,),
        in_specs=[
            pl.BlockSpec((1, block, d_model), lambda i: (0, i, 0)),
        ],
        out_specs=pl.BlockSpec((1, block, d_model), lambda i: (0, i, 0)),
        out_shape=jax.ShapeDtypeStruct((1, seq_len, d_model), embedding.dtype),
        compiler_params=pltpu.CompilerParams(
            dimension_semantics=("parallel",),
        ),
    )(embedding)
